# baseline (speedup 1.0000x reference)
import functools

import jax
import jax.numpy as jnp
from jax import lax
from jax.experimental import pallas as pl
from jax.experimental.pallas import tpu as pltpu
from jax.experimental.pallas import tpu_sc as plsc

BATCH = 16384
OUT_DIM = 128
EMBED_DIM = 64
CAT_DIM = OUT_DIM + EMBED_DIM
NC, NS, L = 2, 16, 16
NW = NC * NS
ROWS_PER_W = BATCH // NW
CHUNK = 128
NCHUNKS = ROWS_PER_W // CHUNK

_RSQRT_MAGIC = 0x5F3759DF


def _inv_norm(s):
    s_safe = jnp.maximum(s, jnp.float32(1.2e-38))
    y = plsc.bitcast(jnp.int32(_RSQRT_MAGIC) - (plsc.bitcast(s_safe, jnp.int32) >> 1),
                     jnp.float32)
    for _ in range(3):
        y = y * (jnp.float32(1.5) - jnp.float32(0.5) * s_safe * y * y)
    n = s * y
    return jnp.float32(1.0) / jnp.maximum(n, jnp.float32(1e-12))


def _sc_body(outs_hbm, ann_hbm, emb_hbm, out_hbm,
             idx_v, ebuf, obuf, catbuf, sem):
    wid = lax.axis_index("s") * NC + lax.axis_index("c")

    for c in range(NCHUNKS):
        base = wid * ROWS_PER_W + c * CHUNK
        pltpu.sync_copy(ann_hbm.at[pl.ds(base, CHUNK)], idx_v)
        gather = pltpu.async_copy(emb_hbm.at[idx_v], ebuf, sem)
        pltpu.sync_copy(outs_hbm.at[pl.ds(base, CHUNK)], obuf)
        gather.wait()

        def row_fn(r, _):
            ov = [obuf[r, pl.ds(j * L, L)] for j in range(OUT_DIM // L)]
            acc = ov[0] * ov[0]
            for v in ov[1:]:
                acc = acc + v * v
            ev = [ebuf[r, pl.ds(j * L, L)] for j in range(EMBED_DIM // L)]
            acc2 = ev[0] * ev[0]
            for v in ev[1:]:
                acc2 = acc2 + v * v
            io = _inv_norm(jnp.broadcast_to(jnp.sum(acc), (L,)))
            ie = _inv_norm(jnp.broadcast_to(jnp.sum(acc2), (L,)))
            for j, v in enumerate(ov):
                catbuf[r, pl.ds(j * L, L)] = v * io
            for j, v in enumerate(ev):
                catbuf[r, pl.ds(OUT_DIM + j * L, L)] = v * ie
            return 0

        lax.fori_loop(0, CHUNK, row_fn, 0)

        pltpu.sync_copy(catbuf, out_hbm.at[pl.ds(base, CHUNK)])


@jax.jit
def _crowd_concat(outputs, annotators, embedding):
    mesh = plsc.VectorSubcoreMesh(core_axis_name="c", subcore_axis_name="s")
    return pl.kernel(
        _sc_body,
        out_type=jax.ShapeDtypeStruct((BATCH, CAT_DIM), jnp.float32),
        mesh=mesh,
        scratch_types=[
            pltpu.VMEM((CHUNK,), jnp.int32),
            pltpu.VMEM((CHUNK, EMBED_DIM), jnp.float32),
            pltpu.VMEM((CHUNK, OUT_DIM), jnp.float32),
            pltpu.VMEM((CHUNK, CAT_DIM), jnp.float32),
            pltpu.SemaphoreType.DMA,
        ],
        compiler_params=pltpu.CompilerParams(
            needs_layout_passes=False, use_tc_tiling_on_sc=False),
    )(outputs, annotators, embedding)


def kernel(outputs, annotators, embedding):
    return _crowd_concat(outputs, annotators, embedding)


# speedup vs baseline: 2.3089x; 2.3089x over previous
import jax
import jax.numpy as jnp
from jax import lax
from jax.experimental import pallas as pl
from jax.experimental.pallas import tpu as pltpu
from jax.experimental.pallas import tpu_sc as plsc

BATCH = 16384
OUT_DIM = 128
EMBED_DIM = 64
CAT_DIM = OUT_DIM + EMBED_DIM
N_ROWS = 1000001
NC, NS, L = 2, 16, 16
NW = NC * NS
ROWS_PER_W = BATCH // NW
CHUNK = 128
GROUPS = CHUNK // L

SLAB = 512
SLABS_PER_W = 61
W_RANGE = SLABS_PER_W * SLAB
EXTRA_SLAB0 = NW * SLABS_PER_W * SLAB
TAIL0 = EXTRA_SLAB0 + SLAB
TAIL_W = N_ROWS - TAIL0
G_ROWS = BATCH + NW

_RSQRT_MAGIC = 0x5F3759DF


def _inv_norm(s):
    s_safe = jnp.maximum(s, jnp.float32(1.2e-38))
    y = plsc.bitcast(
        jnp.int32(_RSQRT_MAGIC) - (plsc.bitcast(s_safe, jnp.int32) >> 1),
        jnp.float32)
    for _ in range(3):
        y = y * (jnp.float32(1.5) - jnp.float32(0.5) * s_safe * y * y)
    n = s * y
    return jnp.float32(1.0) / jnp.maximum(n, jnp.float32(1e-12))


def _scan_body(ann_hbm, emb_t_hbm, g_hbm,
               abuf, whits_r, whits_k, shits_r, shits_k,
               slab_v, tail_v, r2_v, sem):
    wid = lax.axis_index("s") * NC + lax.axis_index("c")
    lo = wid * W_RANGE
    hi = jnp.where(wid == NW - 1, jnp.int32(N_ROWS), lo + W_RANGE)
    lanes = lax.iota(jnp.int32, L)
    dummy = (BATCH + wid) * EMBED_DIM

    def _append(cnt, dst_r, dst_k, rv, kv, m):
        plsc.store_compressed(dst_r.at[pl.ds(cnt, L)], rv, mask=m)
        plsc.store_compressed(dst_k.at[pl.ds(cnt, L)], kv, mask=m)
        npop = plsc.all_reduce_population_count(m)
        return cnt + npop[0]

    def scan_chunk(c8, cnt):
        pltpu.sync_copy(ann_hbm.at[pl.ds(c8 * 2048, 2048)], abuf)

        def scan_group(g, cnt):
            rv = abuf[pl.ds(g * L, L)]
            kv = c8 * 2048 + g * L + lanes
            m = (rv >= lo) & (rv < hi)
            return _append(cnt, whits_r, whits_k, rv, kv, m)

        return lax.fori_loop(0, 2048 // L, scan_group, cnt)

    cnt = lax.fori_loop(0, BATCH // 2048, scan_chunk, jnp.int32(0))

    def do_slab(col0, width, buf):
        pltpu.sync_copy(emb_t_hbm.at[:, pl.ds(col0, width)], buf)

        def filt(h, scnt):
            pos = h * L + lanes
            rv = whits_r[pl.ds(h * L, L)]
            kv = whits_k[pl.ds(h * L, L)]
            m = (pos < cnt) & (rv >= col0) & (rv < col0 + width)
            return _append(scnt, shits_r, shits_k, rv, kv, m)

        scnt = lax.fori_loop(0, (cnt + L - 1) // L, filt, jnp.int32(0))
        nq = (scnt + L - 1) // L

        def extract(q, _):
            pos = q * L + lanes
            vm = pos < scnt
            rv = shits_r[pl.ds(q * L, L)]
            kv = shits_k[pl.ds(q * L, L)]
            rl = jnp.where(vm, rv - col0, 0)
            rowq = (q & 3) * L
            for d in range(EMBED_DIM):
                c = (d + lanes) & (EMBED_DIM - 1)
                vals = plsc.load_gather(buf, [c, rl], mask=vm)
                plsc.store_scatter(r2_v, [(rowq + lanes) * EMBED_DIM + c],
                                   vals, mask=vm)
            ksafe = jnp.where(vm, kv * EMBED_DIM, dummy)
            for j in range(L):
                koff = pl.multiple_of(ksafe[j], EMBED_DIM)
                pltpu.async_copy(
                    r2_v.at[pl.ds((rowq + j) * EMBED_DIM, EMBED_DIM)],
                    g_hbm.at[pl.ds(koff, EMBED_DIM)], sem)
            pltpu.make_async_copy(
                g_hbm.at[pl.ds(0, L * EMBED_DIM)],
                r2_v.at[pl.ds(0, L * EMBED_DIM)], sem).wait()
            return 0

        lax.fori_loop(0, nq, extract, 0)

    def main_slab(s, _):
        do_slab(lo + s * SLAB, SLAB, slab_v)
        return 0

    lax.fori_loop(0, SLABS_PER_W, main_slab, 0)

    @pl.when(wid == NW - 1)
    def _():
        do_slab(jnp.int32(EXTRA_SLAB0), SLAB, slab_v)
        do_slab(jnp.int32(TAIL0), TAIL_W, tail_v)


def _norm_body(outs_t_hbm, g_hbm, out_t_hbm, obuf_t, gbuf, catbuf_t):
    wid = lax.axis_index("s") * NC + lax.axis_index("c")
    lanes = lax.iota(jnp.int32, L)

    for ch in range(ROWS_PER_W // CHUNK):
        base = wid * ROWS_PER_W + ch * CHUNK
        pltpu.sync_copy(outs_t_hbm.at[:, pl.ds(base, CHUNK)], obuf_t)
        pltpu.sync_copy(g_hbm.at[pl.ds(base * EMBED_DIM, CHUNK * EMBED_DIM)],
                        gbuf)

        def group(g, _):
            sl = pl.ds(g * L, L)
            v = obuf_t[0, sl]
            acc = v * v
            for col in range(1, OUT_DIM):
                v = obuf_t[col, sl]
                acc = acc + v * v
            io = _inv_norm(acc)
            rowbase = (g * L + lanes) * EMBED_DIM
            c0 = lanes & (EMBED_DIM - 1)
            vals = plsc.load_gather(gbuf, [rowbase + c0])
            acc2 = vals * vals
            for d in range(1, EMBED_DIM):
                c = (d + lanes) & (EMBED_DIM - 1)
                vals = plsc.load_gather(gbuf, [rowbase + c])
                acc2 = acc2 + vals * vals
            ie = _inv_norm(acc2)
            for col in range(OUT_DIM):
                catbuf_t[col, sl] = obuf_t[col, sl] * io
            for d in range(EMBED_DIM):
                c = (d + lanes) & (EMBED_DIM - 1)
                vals = plsc.load_gather(gbuf, [rowbase + c])
                plsc.store_scatter(catbuf_t, [OUT_DIM + c, g * L + lanes],
                                   vals * ie)
            return 0

        lax.fori_loop(0, GROUPS, group, 0)
        pltpu.sync_copy(catbuf_t, out_t_hbm.at[:, pl.ds(base, CHUNK)])


@jax.jit
def _crowd_concat(outputs, annotators, embedding):
    emb_t = embedding.T
    outs_t = outputs.T
    mesh = plsc.VectorSubcoreMesh(core_axis_name="c", subcore_axis_name="s")
    params = pltpu.CompilerParams(
        needs_layout_passes=False, use_tc_tiling_on_sc=True)

    g1 = pl.kernel(
        _scan_body,
        out_type=jax.ShapeDtypeStruct((G_ROWS * EMBED_DIM,), jnp.float32),
        mesh=mesh,
        scratch_types=[
            pltpu.VMEM((2048,), jnp.int32),
            pltpu.VMEM((BATCH,), jnp.int32),
            pltpu.VMEM((BATCH,), jnp.int32),
            pltpu.VMEM((BATCH,), jnp.int32),
            pltpu.VMEM((BATCH,), jnp.int32),
            pltpu.VMEM((EMBED_DIM, SLAB), jnp.float32),
            pltpu.VMEM((EMBED_DIM, TAIL_W), jnp.float32),
            pltpu.VMEM((4 * L * EMBED_DIM,), jnp.float32),
            pltpu.SemaphoreType.DMA,
        ],
        compiler_params=params,
    )(annotators, emb_t)

    out_t = pl.kernel(
        _norm_body,
        out_type=jax.ShapeDtypeStruct((CAT_DIM, BATCH), jnp.float32),
        mesh=mesh,
        scratch_types=[
            pltpu.VMEM((OUT_DIM, CHUNK), jnp.float32),
            pltpu.VMEM((CHUNK * EMBED_DIM,), jnp.float32),
            pltpu.VMEM((CAT_DIM, CHUNK), jnp.float32),
        ],
        compiler_params=params,
    )(outs_t, g1)

    return out_t.T


def kernel(outputs, annotators, embedding):
    return _crowd_concat(outputs, annotators, embedding)
